# baseline (speedup 1.0000x reference)
import functools

import jax
import jax.numpy as jnp
from jax import lax
from jax.experimental import pallas as pl
from jax.experimental.pallas import tpu as pltpu
from jax.experimental.pallas import tpu_sc as plsc

_STEEPNESS = 12.0
_NB = 16
_RPW = 8
_L = 16


def _mask_sc_body(x1_hbm, w_hbm, mask_hbm, x1_v, w_v, mask_v):
    info = plsc.get_sparse_core_info()
    ns = info.num_subcores
    wid = lax.axis_index("c") * ns + lax.axis_index("s")
    base = wid * _RPW
    _, R, C = x1_hbm.shape
    nv = C // _L

    pltpu.sync_copy(x1_hbm.at[pl.ds(base, _RPW)], x1_v)
    pltpu.sync_copy(w_hbm, w_v)

    idxg = [lax.iota(jnp.int32, _L) + v * _L for v in range(nv)]

    w_vs = [w_v[pl.ds(v * _L, _L)] for v in range(nv)]

    def row_body(b, carry):
        def red_body(r, accs):
            return tuple(
                accs[v] + x1_v[b, r, pl.ds(v * _L, _L)] for v in range(nv))

        accs = lax.fori_loop(
            0, R, red_body,
            tuple(jnp.zeros((_L,), jnp.float32) for _ in range(nv)))
        cvs = [accs[v] * w_vs[v] for v in range(nv)]

        def j_body(lane, ranks):
            ranks = list(ranks)
            lane_v = jnp.zeros((_L,), jnp.int32) + lane
            for u in range(nv):
                jv = lane_v + u * _L
                bj = cvs[u].at[lane_v].get(mode="promise_in_bounds")
                for v in range(nv):
                    gt = bj > cvs[v]
                    tie = (bj == cvs[v]) & (jv < idxg[v])
                    ranks[v] = ranks[v] + jnp.where(gt | tie, 1.0, 0.0)
            return tuple(ranks)

        ranks = lax.fori_loop(
            0, _L, j_body,
            tuple(jnp.zeros((_L,), jnp.float32) for _ in range(nv)),
        )
        for v in range(nv):
            mask_v[b, pl.ds(v * _L, _L)] = jnp.exp(
                (-_STEEPNESS / (C - 1)) * ranks[v])
        return carry

    lax.fori_loop(0, _RPW, row_body, 0)
    pltpu.sync_copy(mask_v, mask_hbm.at[pl.ds(base, _RPW)])


def _apply_body(m_ref, x0_ref, out_ref):
    D = x0_ref.shape[2]
    mask = m_ref[...]
    out_ref[:, :, 0:D - 1] = x0_ref[:, :, 0:D - 1]
    out_ref[:, :, D - 1:D] = (
        x0_ref[:, :, D - 1:D] * mask[:, None, None, None, :]
    )


def kernel(x0, x1, boosting_weights):
    B, C, H, W, D = x0.shape
    xt = lax.transpose(x0, (0, 2, 4, 3, 1))
    x1t = lax.transpose(x1, (0, 2, 1))
    R = x1t.shape[1]

    mask_fn = functools.partial(
        pl.kernel,
        mesh=plsc.VectorSubcoreMesh(core_axis_name="c", subcore_axis_name="s"),
        out_type=jax.ShapeDtypeStruct((B, C), jnp.float32),
        scratch_types=[
            pltpu.VMEM((_RPW, R, C), jnp.float32),
            pltpu.VMEM((C,), jnp.float32),
            pltpu.VMEM((_RPW, C), jnp.float32),
        ],
    )(_mask_sc_body)
    mask = mask_fn(x1t, boosting_weights)

    out = pl.pallas_call(
        _apply_body,
        grid=(B // _NB,),
        in_specs=[
            pl.BlockSpec((_NB, C), lambda i: (i, 0)),
            pl.BlockSpec((_NB, H, D, W, C), lambda i: (i, 0, 0, 0, 0)),
        ],
        out_specs=pl.BlockSpec((_NB, H, D, W, C), lambda i: (i, 0, 0, 0, 0)),
        out_shape=jax.ShapeDtypeStruct((B, H, D, W, C), x0.dtype),
    )(mask, xt)
    return lax.transpose(out, (0, 4, 1, 3, 2))


# speedup vs baseline: 2.1219x; 2.1219x over previous
import functools

import jax
import jax.numpy as jnp
from jax import lax
from jax.experimental import pallas as pl
from jax.experimental.pallas import tpu as pltpu

_STEEPNESS = 12.0
_NB = 16


def _fused_body(x1_ref, w_ref, x0_ref, out_ref):
    nb, R, C = x1_ref.shape
    D = x0_ref.shape[2]

    cr = jnp.sum(x1_ref[...], axis=1) * w_ref[...]

    lane = lax.broadcasted_iota(jnp.int32, (nb, C), 1)
    rank = jnp.zeros((nb, C), jnp.float32)
    for r in range(1, C):
        crj = pltpu.roll(cr, C - r, axis=1)
        gt = crj > cr
        tie = (crj == cr) & (lane >= C - r)
        rank = rank + (gt | tie).astype(jnp.float32)
    mask = jnp.exp((-_STEEPNESS / (C - 1)) * rank)

    out_ref[:, :, 0:D - 1] = x0_ref[:, :, 0:D - 1]
    out_ref[:, :, D - 1:D] = (
        x0_ref[:, :, D - 1:D] * mask[:, None, None, None, :]
    )


def kernel(x0, x1, boosting_weights):
    B, C, H, W, D = x0.shape
    xt = lax.transpose(x0, (0, 2, 4, 3, 1))
    x1t = lax.transpose(x1, (0, 2, 1))
    R = x1t.shape[1]
    w = boosting_weights.reshape(1, C)
    out = pl.pallas_call(
        _fused_body,
        grid=(B // _NB,),
        in_specs=[
            pl.BlockSpec((_NB, R, C), lambda i: (i, 0, 0)),
            pl.BlockSpec((1, C), lambda i: (0, 0)),
            pl.BlockSpec((_NB, H, D, W, C), lambda i: (i, 0, 0, 0, 0)),
        ],
        out_specs=pl.BlockSpec((_NB, H, D, W, C), lambda i: (i, 0, 0, 0, 0)),
        out_shape=jax.ShapeDtypeStruct((B, H, D, W, C), x0.dtype),
    )(x1t, w, xt)
    return lax.transpose(out, (0, 4, 1, 3, 2))
